# baseline (speedup 1.0000x reference)
import functools

import jax
import jax.numpy as jnp
from jax import lax
from jax.experimental import pallas as pl
from jax.experimental.pallas import tpu as pltpu
from jax.experimental.pallas import tpu_sc as plsc

N = 10000
E = 320000
D = 128
NC, NS = 2, 16
NW = NC * NS
RPT = 632
NPAD = NS * RPT
NDEG = 10016
K = 64
NCHUNK = 160
EPAD = NW * NCHUNK * K
PADSRC = 10015


def _normalize_body(f_ref, out_ref):
    f = f_ref[...]
    norm = jnp.sqrt(jnp.sum(f * f, axis=1, keepdims=True))
    out_ref[...] = f / jnp.maximum(norm, 1e-12)


def _spmm_body(fnorm_hbm, src_hbm, dst_hbm, zeros_hbm, out_hbm, deg_hbm,
               sA, dA, sB, dB, rows0, rows1, deg_v, acc,
               sem0, sem1, semia, semib):
    cid = lax.axis_index("c")
    sid = lax.axis_index("s")
    wid = sid * NC + cid

    pltpu.sync_copy(zeros_hbm.at[pl.ds(sid * RPT, RPT)],
                    acc.at[pl.ds(sid * RPT, RPT)])

    zero16 = jnp.zeros((16,), jnp.float32)

    def zbody(i, _):
        deg_v[pl.ds(i * 16, 16)] = zero16
        return ()

    lax.fori_loop(0, NDEG // 16, zbody, ())
    plsc.subcore_barrier()

    ones16 = jnp.ones((16,), jnp.float32)

    def idx_load(ctx_s, ctx_d, chunk, sem):
        pltpu.async_copy(src_hbm.at[wid].at[pl.ds(chunk, 2)], ctx_s, sem)
        pltpu.async_copy(dst_hbm.at[wid].at[pl.ds(chunk, 2)], ctx_d, sem)

    def idx_wait(ctx_s, ctx_d, chunk, sem):
        pltpu.make_async_copy(src_hbm.at[wid].at[pl.ds(chunk, 2)],
                              ctx_s, sem).wait()
        pltpu.make_async_copy(dst_hbm.at[wid].at[pl.ds(chunk, 2)],
                              ctx_d, sem).wait()

    def gather(ctx_d, k, rows, sem):
        pltpu.async_copy(fnorm_hbm.at[ctx_d.at[k]], rows, sem)

    def gwait(ctx_d, k, rows, sem):
        pltpu.make_async_copy(fnorm_hbm.at[ctx_d.at[k]], rows, sem).wait()

    def scat(ctx_s, k, rows):
        pltpu.sync_copy(rows, acc.at[ctx_s.at[k]], add=True)

    def hist(ctx_s):
        for k in range(2):
            for j in range(K // 16):
                idx = ctx_s[k, pl.ds(j * 16, 16)]
                plsc.addupdate_scatter(deg_v, [idx], ones16)

    pltpu.sync_copy(src_hbm.at[wid].at[pl.ds(0, 2)], sA)
    pltpu.sync_copy(dst_hbm.at[wid].at[pl.ds(0, 2)], dA)
    idx_load(sB, dB, 2, semib)
    gather(dA, 0, rows0, sem0)

    def quad(p, _):
        c = 4 * p
        gather(dA, 1, rows1, sem1)
        hist(sA)
        gwait(dA, 0, rows0, sem0)
        scat(sA, 0, rows0)
        idx_wait(sB, dB, c + 2, semib)
        gather(dB, 0, rows0, sem0)
        gwait(dA, 1, rows1, sem1)
        scat(sA, 1, rows1)
        idx_load(sA, dA, c + 4, semia)
        gather(dB, 1, rows1, sem1)
        hist(sB)
        gwait(dB, 0, rows0, sem0)
        scat(sB, 0, rows0)
        idx_wait(sA, dA, c + 4, semia)
        gather(dA, 0, rows0, sem0)
        gwait(dB, 1, rows1, sem1)
        scat(sB, 1, rows1)
        idx_load(sB, dB, c + 6, semib)
        return ()

    lax.fori_loop(0, NCHUNK // 4, quad, ())
    gwait(dA, 0, rows0, sem0)
    idx_wait(sB, dB, NCHUNK + 2, semib)

    pltpu.sync_copy(deg_v, deg_hbm.at[wid])
    plsc.subcore_barrier()
    pltpu.sync_copy(acc.at[pl.ds(sid * RPT, RPT)],
                    out_hbm.at[cid].at[pl.ds(sid * RPT, RPT)])


def _finish_body(part_ref, deg_ref, fnorm_ref, out_ref):
    acc = part_ref[0] + part_ref[1]
    s = acc[:N, :]
    fn = fnorm_ref[...]
    sim = jnp.sum(fn * s, axis=1, keepdims=True)
    deg = lax.dot_general(deg_ref[...], jnp.ones((NW, 1), jnp.float32),
                          (((0,), (0,)), ((), ())),
                          preferred_element_type=jnp.float32,
                          precision=lax.Precision.HIGHEST)
    deg = deg[:N, :]
    deg = jnp.where(deg == 0.0, 1.0, deg)
    score = (sim / deg + 1.0) * 0.5
    out_ref[...] = jnp.clip(score, 0.0, 1.0)


def kernel(features, edge_index):
    fnorm = pl.pallas_call(
        _normalize_body,
        out_shape=jax.ShapeDtypeStruct((N, D), jnp.float32),
    )(features)

    pad = EPAD - E
    src = jnp.concatenate([edge_index[0],
                           jnp.full((pad,), PADSRC, jnp.int32)])
    dst = jnp.concatenate([edge_index[1], jnp.zeros((pad,), jnp.int32)])
    dummy = jnp.zeros((NW, 4, K), jnp.int32)
    src3 = jnp.concatenate([src.reshape(NW, NCHUNK, K), dummy], axis=1)
    dst3 = jnp.concatenate([dst.reshape(NW, NCHUNK, K), dummy], axis=1)
    zeros = jnp.zeros((NPAD, D), jnp.float32)

    mesh = plsc.VectorSubcoreMesh(core_axis_name="c", subcore_axis_name="s")
    spmm = functools.partial(
        pl.kernel,
        out_type=(
            jax.ShapeDtypeStruct((NC, NPAD, D), jnp.float32),
            jax.ShapeDtypeStruct((NW, NDEG), jnp.float32),
        ),
        mesh=mesh,
        compiler_params=pltpu.CompilerParams(needs_layout_passes=False),
        scratch_types=[
            pltpu.VMEM((2, K), jnp.int32),
            pltpu.VMEM((2, K), jnp.int32),
            pltpu.VMEM((2, K), jnp.int32),
            pltpu.VMEM((2, K), jnp.int32),
            pltpu.VMEM((K, D), jnp.float32),
            pltpu.VMEM((K, D), jnp.float32),
            pltpu.VMEM((NDEG,), jnp.float32),
            pltpu.VMEM_SHARED((NPAD, D), jnp.float32),
            pltpu.SemaphoreType.DMA,
            pltpu.SemaphoreType.DMA,
            pltpu.SemaphoreType.DMA,
            pltpu.SemaphoreType.DMA,
        ],
    )(_spmm_body)
    partials, deg_part = spmm(fnorm, src3, dst3, zeros)

    scores = pl.pallas_call(
        _finish_body,
        out_shape=jax.ShapeDtypeStruct((N, 1), jnp.float32),
    )(partials, deg_part, fnorm)
    return scores[:, 0]


# speedup vs baseline: 1.7204x; 1.7204x over previous
import functools

import jax
import jax.numpy as jnp
from jax import lax
from jax.experimental import pallas as pl
from jax.experimental.pallas import tpu as pltpu
from jax.experimental.pallas import tpu_sc as plsc

N = 10000
E = 320000
D = 128
NC, NS = 2, 16
NW = NC * NS
RPT = 632
NPAD = NS * RPT
NDEG = 10016
K = 80
NCHUNK = E // (NW * K)


def _normalize_body(f_ref, out_ref):
    f = f_ref[...]
    norm = jnp.sqrt(jnp.sum(f * f, axis=1, keepdims=True))
    out_ref[...] = f / jnp.maximum(norm, 1e-12)


def _spmm_body(fnorm_hbm, src_hbm, dst_hbm, zeros_hbm, out_hbm, deg_hbm,
               idx_s, idx_d, rows, deg_v, acc):
    cid = lax.axis_index("c")
    sid = lax.axis_index("s")
    wid = sid * NC + cid

    pltpu.sync_copy(zeros_hbm.at[pl.ds(sid * RPT, RPT)],
                    acc.at[pl.ds(sid * RPT, RPT)])

    zero16 = jnp.zeros((16,), jnp.float32)

    def zbody(i, _):
        deg_v[pl.ds(i * 16, 16)] = zero16
        return ()

    lax.fori_loop(0, NDEG // 16, zbody, ())
    plsc.subcore_barrier()

    ones16 = jnp.ones((16,), jnp.float32)

    def body(c, _):
        pltpu.sync_copy(src_hbm.at[wid].at[pl.ds(c, 1)], idx_s)
        pltpu.sync_copy(dst_hbm.at[wid].at[pl.ds(c, 1)], idx_d)
        pltpu.sync_copy(fnorm_hbm.at[idx_d.at[0]], rows)
        pltpu.sync_copy(rows, acc.at[idx_s.at[0]], add=True)
        for j in range(K // 16):
            idx = idx_s[0, pl.ds(j * 16, 16)]
            plsc.addupdate_scatter(deg_v, [idx], ones16)
        return ()

    lax.fori_loop(0, NCHUNK, body, ())

    pltpu.sync_copy(deg_v, deg_hbm.at[wid])
    plsc.subcore_barrier()
    pltpu.sync_copy(acc.at[pl.ds(sid * RPT, RPT)],
                    out_hbm.at[cid].at[pl.ds(sid * RPT, RPT)])


def _finish_body(part_ref, deg_ref, fnorm_ref, out_ref):
    acc = part_ref[0] + part_ref[1]
    s = acc[:N, :]
    fn = fnorm_ref[...]
    sim = jnp.sum(fn * s, axis=1, keepdims=True)
    deg = lax.dot_general(deg_ref[...], jnp.ones((NW, 1), jnp.float32),
                          (((0,), (0,)), ((), ())),
                          preferred_element_type=jnp.float32,
                          precision=lax.Precision.HIGHEST)
    deg = deg[:N, :]
    deg = jnp.where(deg == 0.0, 1.0, deg)
    score = (sim / deg + 1.0) * 0.5
    out_ref[...] = jnp.clip(score, 0.0, 1.0)


def kernel(features, edge_index):
    fnorm = pl.pallas_call(
        _normalize_body,
        out_shape=jax.ShapeDtypeStruct((N, D), jnp.float32),
    )(features)

    src3 = edge_index[0].reshape(NW, NCHUNK, K)
    dst3 = edge_index[1].reshape(NW, NCHUNK, K)
    zeros = jnp.zeros((NPAD, D), jnp.float32)

    mesh = plsc.VectorSubcoreMesh(core_axis_name="c", subcore_axis_name="s")
    spmm = functools.partial(
        pl.kernel,
        out_type=(
            jax.ShapeDtypeStruct((NC, NPAD, D), jnp.float32),
            jax.ShapeDtypeStruct((NW, NDEG), jnp.float32),
        ),
        mesh=mesh,
        compiler_params=pltpu.CompilerParams(needs_layout_passes=False),
        scratch_types=[
            pltpu.VMEM((1, K), jnp.int32),
            pltpu.VMEM((1, K), jnp.int32),
            pltpu.VMEM((K, D), jnp.float32),
            pltpu.VMEM((NDEG,), jnp.float32),
            pltpu.VMEM_SHARED((NPAD, D), jnp.float32),
        ],
    )(_spmm_body)
    partials, deg_part = spmm(fnorm, src3, dst3, zeros)

    scores = pl.pallas_call(
        _finish_body,
        out_shape=jax.ShapeDtypeStruct((N, 1), jnp.float32),
    )(partials, deg_part, fnorm)
    return scores[:, 0]


# speedup vs baseline: 1.7446x; 1.0141x over previous
import functools

import jax
import jax.numpy as jnp
from jax import lax
from jax.experimental import pallas as pl
from jax.experimental.pallas import tpu as pltpu
from jax.experimental.pallas import tpu_sc as plsc

N = 10000
E = 320000
D = 128
NC, NS = 2, 16
NW = NC * NS
RPT = 632
NPAD = NS * RPT
NDEG = 10016
K = 128
NCHUNK = 79
EPAD = NW * NCHUNK * K
PADSRC = 10008


def _normalize_body(f_ref, out_ref):
    f = f_ref[...]
    norm = jnp.sqrt(jnp.sum(f * f, axis=1, keepdims=True))
    out_ref[...] = f / jnp.maximum(norm, 1e-12)


def _spmm_body(fnorm_hbm, src_hbm, dst_hbm, zeros_hbm, out_hbm, deg_hbm,
               idx_s, idx_d, rows0, rows1, deg_v, acc, sem0, sem1):
    cid = lax.axis_index("c")
    sid = lax.axis_index("s")
    wid = sid * NC + cid

    pltpu.async_copy(src_hbm.at[wid], idx_s, sem0)
    pltpu.async_copy(dst_hbm.at[wid], idx_d, sem1)
    pltpu.sync_copy(zeros_hbm.at[pl.ds(sid * RPT, RPT)],
                    acc.at[pl.ds(sid * RPT, RPT)])

    zero16 = jnp.zeros((16,), jnp.float32)

    def zbody(i, _):
        deg_v[pl.ds(i * 16, 16)] = zero16
        return ()

    lax.fori_loop(0, NDEG // 16, zbody, ())
    pltpu.make_async_copy(src_hbm.at[wid], idx_s, sem0).wait()
    pltpu.make_async_copy(dst_hbm.at[wid], idx_d, sem1).wait()
    plsc.subcore_barrier()

    ones16 = jnp.ones((16,), jnp.float32)

    def gather(c, rows, sem):
        pltpu.async_copy(fnorm_hbm.at[idx_d.at[c]], rows, sem)

    def gwait(c, rows, sem):
        pltpu.make_async_copy(fnorm_hbm.at[idx_d.at[c]], rows, sem).wait()

    def hist(c):
        for j in range(K // 16):
            idx = idx_s[c, pl.ds(j * 16, 16)]
            plsc.addupdate_scatter(deg_v, [idx], ones16)

    def body(c, _):
        pltpu.sync_copy(fnorm_hbm.at[idx_d.at[c]], rows0)
        hist(c)
        pltpu.sync_copy(rows0, acc.at[idx_s.at[c]], add=True)
        return ()

    lax.fori_loop(0, NCHUNK, body, ())

    pltpu.sync_copy(deg_v, deg_hbm.at[wid])
    plsc.subcore_barrier()
    pltpu.sync_copy(acc.at[pl.ds(sid * RPT, RPT)],
                    out_hbm.at[cid].at[pl.ds(sid * RPT, RPT)])


def _finish_body(part_ref, deg_ref, fnorm_ref, out_ref):
    acc = part_ref[0] + part_ref[1]
    s = acc[:N, :]
    fn = fnorm_ref[...]
    sim = jnp.sum(fn * s, axis=1, keepdims=True)
    deg = lax.dot_general(deg_ref[...], jnp.ones((NW, 1), jnp.float32),
                          (((0,), (0,)), ((), ())),
                          preferred_element_type=jnp.float32,
                          precision=lax.Precision.HIGHEST)
    deg = deg[:N, :]
    deg = jnp.where(deg == 0.0, 1.0, deg)
    score = (sim / deg + 1.0) * 0.5
    out_ref[...] = jnp.clip(score, 0.0, 1.0)


def kernel(features, edge_index):
    fnorm = pl.pallas_call(
        _normalize_body,
        out_shape=jax.ShapeDtypeStruct((N, D), jnp.float32),
    )(features)

    pad = EPAD - E
    src3 = jnp.concatenate(
        [edge_index[0], jnp.full((pad,), PADSRC, jnp.int32)]
    ).reshape(NW, NCHUNK, K)
    dst3 = jnp.concatenate(
        [edge_index[1], jnp.zeros((pad,), jnp.int32)]
    ).reshape(NW, NCHUNK, K)
    zeros = jnp.zeros((NPAD, D), jnp.float32)

    mesh = plsc.VectorSubcoreMesh(core_axis_name="c", subcore_axis_name="s")
    spmm = functools.partial(
        pl.kernel,
        out_type=(
            jax.ShapeDtypeStruct((NC, NPAD, D), jnp.float32),
            jax.ShapeDtypeStruct((NW, NDEG), jnp.float32),
        ),
        mesh=mesh,
        compiler_params=pltpu.CompilerParams(needs_layout_passes=False),
        scratch_types=[
            pltpu.VMEM((NCHUNK, K), jnp.int32),
            pltpu.VMEM((NCHUNK, K), jnp.int32),
            pltpu.VMEM((K, D), jnp.float32),
            pltpu.VMEM((16,), jnp.float32),
            pltpu.VMEM((NDEG,), jnp.float32),
            pltpu.VMEM_SHARED((NPAD, D), jnp.float32),
            pltpu.SemaphoreType.DMA,
            pltpu.SemaphoreType.DMA,
        ],
    )(_spmm_body)
    partials, deg_part = spmm(fnorm, src3, dst3, zeros)

    scores = pl.pallas_call(
        _finish_body,
        out_shape=jax.ShapeDtypeStruct((N, 1), jnp.float32),
    )(partials, deg_part, fnorm)
    return scores[:, 0]


# speedup vs baseline: 2.0883x; 1.1970x over previous
import functools

import jax
import jax.numpy as jnp
from jax import lax
from jax.experimental import pallas as pl
from jax.experimental.pallas import tpu as pltpu
from jax.experimental.pallas import tpu_sc as plsc

N = 10000
E = 320000
D = 128
NC, NS = 2, 16
NW = NC * NS
RPT = 632
NPAD = NS * RPT
NDEG = 10016
K = 128
NCHUNK = 79
EPAD = NW * NCHUNK * K
PADSRC = 10008


def _normalize_body(f_ref, out_ref):
    f = f_ref[...]
    norm = jnp.sqrt(jnp.sum(f * f, axis=1, keepdims=True))
    out_ref[...] = f / jnp.maximum(norm, 1e-12)


def _hist_body(src_hbm, deg_hbm, idx_s, deg_v, sem):
    cid = lax.axis_index("c")
    sid = lax.axis_index("s")
    wid = sid * NC + cid

    pltpu.async_copy(src_hbm.at[wid], idx_s, sem)

    zero16 = jnp.zeros((16,), jnp.float32)

    def zbody(i, _):
        deg_v[pl.ds(i * 16, 16)] = zero16
        return ()

    lax.fori_loop(0, NDEG // 16, zbody, ())
    pltpu.make_async_copy(src_hbm.at[wid], idx_s, sem).wait()

    ones16 = jnp.ones((16,), jnp.float32)

    def body(c, _):
        for j in range(K // 16):
            idx = idx_s[c, pl.ds(j * 16, 16)]
            plsc.addupdate_scatter(deg_v, [idx], ones16)
        return ()

    lax.fori_loop(0, NCHUNK, body, ())
    pltpu.sync_copy(deg_v, deg_hbm.at[wid])


def _spmm_body(fnorm_hbm, src_hbm, dst_hbm, zeros_hbm, out_hbm,
               idx_d, srcA, srcB, rows0, rows1, acc,
               sem0, sem1, semA, semB):
    cid = lax.axis_index("c")
    sid = lax.axis_index("s")
    wid = sid * NC + cid

    pltpu.async_copy(dst_hbm.at[wid], idx_d, sem0)
    pltpu.async_copy(src_hbm.at[wid].at[pl.ds(0, 1)], srcA, semA)
    pltpu.async_copy(src_hbm.at[wid].at[pl.ds(1, 1)], srcB, semB)
    pltpu.sync_copy(zeros_hbm.at[pl.ds(sid * RPT, RPT)],
                    acc.at[pl.ds(sid * RPT, RPT)])
    pltpu.make_async_copy(dst_hbm.at[wid], idx_d, sem0).wait()
    plsc.subcore_barrier()

    def gather(c, rows, sem):
        pltpu.async_copy(fnorm_hbm.at[idx_d.at[c]], rows, sem)

    def gwait(c, rows, sem):
        pltpu.make_async_copy(fnorm_hbm.at[idx_d.at[c]], rows, sem).wait()

    def srcload(buf, c, sem):
        pltpu.async_copy(src_hbm.at[wid].at[pl.ds(c, 1)], buf, sem)

    def srcwait(buf, c, sem):
        pltpu.make_async_copy(src_hbm.at[wid].at[pl.ds(c, 1)], buf,
                              sem).wait()

    def scat(buf, rows):
        pltpu.sync_copy(rows, acc.at[buf.at[0]], add=True)

    gather(0, rows0, sem0)
    last = NCHUNK - 1

    def body(p, _):
        c = 2 * p
        gather(c + 1, rows1, sem1)
        gwait(c, rows0, sem0)
        srcwait(srcA, c, semA)
        scat(srcA, rows0)
        srcload(srcA, jnp.minimum(c + 2, last), semA)
        gather(c + 2, rows0, sem0)
        gwait(c + 1, rows1, sem1)
        srcwait(srcB, c + 1, semB)
        scat(srcB, rows1)
        srcload(srcB, jnp.minimum(c + 3, last), semB)
        return ()

    lax.fori_loop(0, (NCHUNK - 1) // 2, body, ())
    gwait(last, rows0, sem0)
    srcwait(srcA, last, semA)
    scat(srcA, rows0)
    srcwait(srcB, last, semB)

    plsc.subcore_barrier()
    pltpu.sync_copy(acc.at[pl.ds(sid * RPT, RPT)],
                    out_hbm.at[cid].at[pl.ds(sid * RPT, RPT)])


def _finish_body(part_ref, deg_ref, fnorm_ref, out_ref):
    acc = part_ref[0] + part_ref[1]
    s = acc[:N, :]
    fn = fnorm_ref[...]
    sim = jnp.sum(fn * s, axis=1, keepdims=True)
    deg = lax.dot_general(deg_ref[...], jnp.ones((NW, 1), jnp.float32),
                          (((0,), (0,)), ((), ())),
                          preferred_element_type=jnp.float32,
                          precision=lax.Precision.HIGHEST)
    deg = deg[:N, :]
    deg = jnp.where(deg == 0.0, 1.0, deg)
    score = (sim / deg + 1.0) * 0.5
    out_ref[...] = jnp.clip(score, 0.0, 1.0)


def kernel(features, edge_index):
    fnorm = pl.pallas_call(
        _normalize_body,
        out_shape=jax.ShapeDtypeStruct((N, D), jnp.float32),
    )(features)

    pad = EPAD - E
    src3 = jnp.concatenate(
        [edge_index[0], jnp.full((pad,), PADSRC, jnp.int32)]
    ).reshape(NW, NCHUNK, K)
    dst3 = jnp.concatenate(
        [edge_index[1], jnp.zeros((pad,), jnp.int32)]
    ).reshape(NW, NCHUNK, K)
    zeros = jnp.zeros((NPAD, D), jnp.float32)

    mesh = plsc.VectorSubcoreMesh(core_axis_name="c", subcore_axis_name="s")

    hist = functools.partial(
        pl.kernel,
        out_type=jax.ShapeDtypeStruct((NW, NDEG), jnp.float32),
        mesh=mesh,
        compiler_params=pltpu.CompilerParams(needs_layout_passes=False),
        scratch_types=[
            pltpu.VMEM((NCHUNK, K), jnp.int32),
            pltpu.VMEM((NDEG,), jnp.float32),
            pltpu.SemaphoreType.DMA,
        ],
    )(_hist_body)
    deg_part = hist(src3)

    spmm = functools.partial(
        pl.kernel,
        out_type=jax.ShapeDtypeStruct((NC, NPAD, D), jnp.float32),
        mesh=mesh,
        compiler_params=pltpu.CompilerParams(needs_layout_passes=False),
        scratch_types=[
            pltpu.VMEM((NCHUNK, K), jnp.int32),
            pltpu.VMEM((1, K), jnp.int32),
            pltpu.VMEM((1, K), jnp.int32),
            pltpu.VMEM((K, D), jnp.float32),
            pltpu.VMEM((K, D), jnp.float32),
            pltpu.VMEM_SHARED((NPAD, D), jnp.float32),
            pltpu.SemaphoreType.DMA,
            pltpu.SemaphoreType.DMA,
            pltpu.SemaphoreType.DMA,
            pltpu.SemaphoreType.DMA,
        ],
    )(_spmm_body)
    partials = spmm(fnorm, src3, dst3, zeros)

    scores = pl.pallas_call(
        _finish_body,
        out_shape=jax.ShapeDtypeStruct((N, 1), jnp.float32),
    )(partials, deg_part, fnorm)
    return scores[:, 0]
